# Initial kernel scaffold; baseline (speedup 1.0000x reference)
import jax
import jax.numpy as jnp
from jax.experimental import pallas as pl


def kernel(x, weight, bias):
    raise NotImplementedError("write your pallas kernel here")


# speedup vs baseline: 1.2611x; 1.2611x over previous
import jax
import jax.numpy as jnp
from jax.experimental import pallas as pl
from jax.experimental.pallas import tpu as pltpu

_VMEM_LIMIT = 64 * 1024 * 1024


def _dsconv_kernel(xm_ref, xh_ref, w_ref, b_ref, o_ref):
    TR = xm_ref.shape[1]
    Wo = xm_ref.shape[3]
    two_c = xm_ref.shape[4]
    c = two_c // 2
    m = TR * Wo

    xm = xm_ref[0]
    rows0 = xm[:, 0]
    rows1 = xm[:, 1]
    halo = xh_ref[0, 0, 0]
    is_last = pl.program_id(1) == pl.num_programs(1) - 1
    halo = jnp.where(is_last, jnp.bfloat16(0), halo)
    if TR > 1:
        rows2 = jnp.concatenate([rows0[1:], halo[None]], axis=0)
    else:
        rows2 = halo[None]

    pieces = []
    for rows in (rows0, rows1, rows2):
        pieces.append(rows.reshape(m, two_c))
        s = jnp.concatenate(
            [rows[:, 1:, :c], jnp.zeros((TR, 1, c), jnp.bfloat16)], axis=1)
        pieces.append(s.reshape(m, c))
    lhs = jnp.concatenate(pieces, axis=-1)

    acc = jax.lax.dot_general(
        w_ref[...], lhs, (((1,), (1,)), ((), ())),
        preferred_element_type=jnp.float32)
    o_ref[0] = acc + b_ref[...]


@jax.jit
def kernel(x, weight, bias):
    B, C, H, W = x.shape
    Cout = weight.shape[0]
    Ho, Wo = H // 2, W // 2
    TR = 16
    nb = Ho // TR

    xt = jnp.transpose(x, (0, 2, 3, 1)).astype(jnp.bfloat16)
    xv = xt.reshape(B, Ho, 2, Wo, 2 * C)

    w9 = jnp.transpose(weight, (0, 2, 3, 1)).reshape(Cout, 9 * C)
    w9 = w9.astype(jnp.bfloat16)
    b_col = bias.reshape(Cout, 1).astype(jnp.float32)

    m, k = B * Ho * Wo, 9 * C
    cost = pl.CostEstimate(
        flops=int(2 * m * k * Cout),
        transcendentals=0,
        bytes_accessed=int(xv.size * 2 + w9.size * 2 + m * Cout * 4))

    out = pl.pallas_call(
        _dsconv_kernel,
        out_shape=jax.ShapeDtypeStruct((B, Cout, Ho * Wo), jnp.float32),
        grid_spec=pltpu.PrefetchScalarGridSpec(
            num_scalar_prefetch=0,
            grid=(B, nb),
            in_specs=[
                pl.BlockSpec((1, TR, 2, Wo, 2 * C),
                             lambda b, i: (b, i, 0, 0, 0)),
                pl.BlockSpec((1, 1, 2, Wo, 2 * C),
                             lambda b, i: (b, jnp.minimum((i + 1) * TR,
                                                          Ho - 1), 0, 0, 0)),
                pl.BlockSpec((Cout, 9 * C), lambda b, i: (0, 0)),
                pl.BlockSpec((Cout, 1), lambda b, i: (0, 0)),
            ],
            out_specs=pl.BlockSpec((1, Cout, TR * Wo), lambda b, i: (b, 0, i)),
        ),
        compiler_params=pltpu.CompilerParams(
            dimension_semantics=("parallel", "parallel"),
            vmem_limit_bytes=_VMEM_LIMIT),
        cost_estimate=cost,
    )(xv, xv, w9, b_col)

    return out.reshape(B, Cout, Ho, Wo)


# speedup vs baseline: 1.6845x; 1.3357x over previous
import jax
import jax.numpy as jnp
from jax.experimental import pallas as pl
from jax.experimental.pallas import tpu as pltpu

_VMEM_LIMIT = 64 * 1024 * 1024


def _dsconv_kernel(xm_ref, xh_ref, w_ref, b_ref, o_ref):
    C = xm_ref.shape[1]
    W = xh_ref.shape[2]
    TR = xm_ref.shape[2] // (2 * W)
    Wo = W // 2
    m = TR * Wo

    xb = xm_ref[0].astype(jnp.bfloat16)
    xt = jnp.transpose(xb, (1, 0))
    xtp = xt.reshape(TR, 2, Wo, 2 * C)
    rows0 = xtp[:, 0]
    rows1 = xtp[:, 1]

    xh = xh_ref[0].astype(jnp.bfloat16)
    halo = jnp.transpose(xh, (1, 0)).reshape(1, Wo, 2 * C)
    is_last = pl.program_id(1) == pl.num_programs(1) - 1
    halo = jnp.where(is_last, jnp.bfloat16(0), halo)
    if TR > 1:
        rows2 = jnp.concatenate([rows0[1:], halo], axis=0)
    else:
        rows2 = halo

    pieces = []
    for rows in (rows0, rows1, rows2):
        pieces.append(rows.reshape(m, 2 * C))
        s = jnp.concatenate(
            [rows[:, 1:, :C], jnp.zeros((TR, 1, C), jnp.bfloat16)], axis=1)
        pieces.append(s.reshape(m, C))
    lhs = jnp.concatenate(pieces, axis=-1)

    acc = jax.lax.dot_general(
        w_ref[...], lhs, (((1,), (1,)), ((), ())),
        preferred_element_type=jnp.float32)
    o_ref[0] = acc + b_ref[...]


@jax.jit
def kernel(x, weight, bias):
    B, C, H, W = x.shape
    Cout = weight.shape[0]
    Ho, Wo = H // 2, W // 2
    TR = 16
    nb = Ho // TR

    xf = x.reshape(B, C, H * W)

    w9 = jnp.transpose(weight, (0, 2, 3, 1)).reshape(Cout, 9 * C)
    w9 = w9.astype(jnp.bfloat16)
    b_col = bias.reshape(Cout, 1).astype(jnp.float32)

    m, k = B * Ho * Wo, 9 * C
    cost = pl.CostEstimate(
        flops=int(2 * m * k * Cout),
        transcendentals=0,
        bytes_accessed=int(x.size * 4 + w9.size * 2 + m * Cout * 4))

    out = pl.pallas_call(
        _dsconv_kernel,
        out_shape=jax.ShapeDtypeStruct((B, Cout, Ho * Wo), jnp.float32),
        grid_spec=pltpu.PrefetchScalarGridSpec(
            num_scalar_prefetch=0,
            grid=(B, nb),
            in_specs=[
                pl.BlockSpec((1, C, 2 * TR * W), lambda b, i: (b, 0, i)),
                pl.BlockSpec((1, C, W),
                             lambda b, i: (b, 0, jnp.minimum(2 * (i + 1) * TR,
                                                             H - 1))),
                pl.BlockSpec((Cout, 9 * C), lambda b, i: (0, 0)),
                pl.BlockSpec((Cout, 1), lambda b, i: (0, 0)),
            ],
            out_specs=pl.BlockSpec((1, Cout, TR * Wo), lambda b, i: (b, 0, i)),
        ),
        compiler_params=pltpu.CompilerParams(
            dimension_semantics=("parallel", "parallel"),
            vmem_limit_bytes=_VMEM_LIMIT),
        cost_estimate=cost,
    )(xf, xf, w9, b_col)

    return out.reshape(B, Cout, Ho, Wo)


# speedup vs baseline: 3.1471x; 1.8683x over previous
import jax
import jax.numpy as jnp
from jax.experimental import pallas as pl
from jax.experimental.pallas import tpu as pltpu

_VMEM_LIMIT = 64 * 1024 * 1024


def _dsconv_kernel(xm_ref, w_ref, b_ref, o_ref):
    C = xm_ref.shape[1]
    H = xm_ref.shape[2]
    W = xm_ref.shape[3]
    Ho, Wo = H // 2, W // 2
    m = Ho * Wo

    xb = xm_ref[0].astype(jnp.bfloat16)
    xt = jnp.transpose(xb, (1, 2, 0))
    xtp = xt.reshape(Ho, 2, Wo, 2 * C)
    rows0 = xtp[:, 0]
    rows1 = xtp[:, 1]
    rows2 = jnp.concatenate(
        [rows0[1:], jnp.zeros((1, Wo, 2 * C), jnp.bfloat16)], axis=0)

    pieces = []
    for rows in (rows0, rows1, rows2):
        pieces.append(rows.reshape(m, 2 * C))
        s = jnp.concatenate(
            [rows[:, 1:, :C], jnp.zeros((Ho, 1, C), jnp.bfloat16)], axis=1)
        pieces.append(s.reshape(m, C))
    lhs = jnp.concatenate(pieces, axis=-1)

    acc = jax.lax.dot_general(
        w_ref[...], lhs, (((1,), (1,)), ((), ())),
        preferred_element_type=jnp.float32)
    o_ref[0] = acc + b_ref[...]


@jax.jit
def kernel(x, weight, bias):
    B, C, H, W = x.shape
    Cout = weight.shape[0]
    Ho, Wo = H // 2, W // 2

    w9 = jnp.transpose(weight, (0, 2, 3, 1)).reshape(Cout, 9 * C)
    w9 = w9.astype(jnp.bfloat16)
    b_col = bias.reshape(Cout, 1).astype(jnp.float32)

    m, k = B * Ho * Wo, 9 * C
    cost = pl.CostEstimate(
        flops=int(2 * m * k * Cout),
        transcendentals=0,
        bytes_accessed=int(x.size * 4 + w9.size * 2 + m * Cout * 4))

    out = pl.pallas_call(
        _dsconv_kernel,
        out_shape=jax.ShapeDtypeStruct((B, Cout, Ho * Wo), jnp.float32),
        grid_spec=pltpu.PrefetchScalarGridSpec(
            num_scalar_prefetch=0,
            grid=(B,),
            in_specs=[
                pl.BlockSpec((1, C, H, W), lambda b: (b, 0, 0, 0)),
                pl.BlockSpec((Cout, 9 * C), lambda b: (0, 0)),
                pl.BlockSpec((Cout, 1), lambda b: (0, 0)),
            ],
            out_specs=pl.BlockSpec((1, Cout, Ho * Wo), lambda b: (b, 0, 0)),
        ),
        compiler_params=pltpu.CompilerParams(
            dimension_semantics=("parallel",),
            vmem_limit_bytes=_VMEM_LIMIT),
        cost_estimate=cost,
    )(x, w9, b_col)

    return out.reshape(B, Cout, Ho, Wo)
